# Initial kernel scaffold; baseline (speedup 1.0000x reference)
import jax
import jax.numpy as jnp
from jax.experimental import pallas as pl


def kernel(at_no, pos, edge_index, batch, params):
    raise NotImplementedError("write your pallas kernel here")


# speedup vs baseline: 9.8486x; 9.8486x over previous
import functools

import jax
import jax.numpy as jnp
from jax import lax
from jax.experimental import pallas as pl
from jax.experimental.pallas import tpu as pltpu
from jax.experimental.pallas import tpu_sc as plsc

N = 10000
E = 160000
D = 128
NB = 20
HID = 64
NG = 64
CUTOFF = 5.0

NP = 10240
EP = 163840
NW = 32

F32 = jnp.float32


def _sc_gather(Vrows, Dcols, B, chunk):
    per = B // NW
    iters = per // chunk
    mesh = plsc.VectorSubcoreMesh(core_axis_name="c", subcore_axis_name="s")

    @functools.partial(
        pl.kernel,
        out_type=jax.ShapeDtypeStruct((B, Dcols), F32),
        mesh=mesh,
        scratch_types=[
            pltpu.VMEM((chunk,), jnp.int32),
            pltpu.VMEM((chunk, Dcols), F32),
            pltpu.SemaphoreType.DMA,
        ],
    )
    def k(table_hbm, idx_hbm, out_hbm, idx_v, rows_v, sem):
        wid = lax.axis_index("s") * 2 + lax.axis_index("c")
        base = wid * per

        def body(i, carry):
            off = base + i * chunk
            pltpu.sync_copy(idx_hbm.at[pl.ds(off, chunk)], idx_v)
            pltpu.async_copy(table_hbm.at[idx_v], rows_v, sem).wait()
            pltpu.sync_copy(rows_v, out_hbm.at[pl.ds(off, chunk)])
            return carry

        lax.fori_loop(0, iters, body, 0)

    return k


def _sc_scatter_add(B, chunk):
    per = B // NW
    iters = per // chunk
    zper = NP // 16
    mesh = plsc.VectorSubcoreMesh(core_axis_name="c", subcore_axis_name="s")

    @functools.partial(
        pl.kernel,
        out_type=jax.ShapeDtypeStruct((2, NP, 128), F32),
        mesh=mesh,
        scratch_types=[
            pltpu.VMEM((chunk,), jnp.int32),
            pltpu.VMEM((chunk, 128), F32),
            pltpu.VMEM_SHARED((NP, 128), F32),
        ],
    )
    def k(rows_hbm, dst_hbm, zeros_hbm, out_hbm, idx_v, rows_v, acc):
        c = lax.axis_index("c")
        s = lax.axis_index("s")
        wid = s * 2 + c
        base = wid * per
        pltpu.sync_copy(zeros_hbm.at[pl.ds(s * zper, zper)],
                        acc.at[pl.ds(s * zper, zper)])
        plsc.subcore_barrier()

        def body(i, carry):
            off = base + i * chunk
            pltpu.sync_copy(dst_hbm.at[pl.ds(off, chunk)], idx_v)
            pltpu.sync_copy(rows_hbm.at[pl.ds(off, chunk)], rows_v)
            pltpu.sync_copy(rows_v, acc.at[idx_v], add=True)
            return carry

        lax.fori_loop(0, iters, body, 0)
        plsc.subcore_barrier()
        pltpu.sync_copy(acc.at[pl.ds(s * zper, zper)],
                        out_hbm.at[c, pl.ds(s * zper, zper)])

    return k


def _silu(x):
    return x * jax.nn.sigmoid(x)


def _geom_body(ps_ref, pd_ref, out_ref):
    vec = pd_ref[...] - ps_ref[...]
    v0 = vec[:, 0:1]
    v1 = vec[:, 1:2]
    v2 = vec[:, 2:3]
    d2 = v0 * v0 + v1 * v1 + v2 * v2 + 1e-12
    dist = jnp.sqrt(d2)
    inv = 1.0 / dist
    coli = lax.broadcasted_iota(jnp.int32, out_ref.shape, 1)
    narr = coli.astype(F32) + 1.0
    rbf = jnp.sqrt(2.0 / CUTOFF) * jnp.sin(narr * (jnp.pi / CUTOFF) * dist) * inv
    env = jnp.where(dist < CUTOFF, 0.5 * (jnp.cos((jnp.pi / CUTOFF) * dist) + 1.0), 0.0)
    rbfe = rbf * env
    out = jnp.where(coli < 20, rbfe,
                    jnp.where(coli == 20, v0 * inv,
                              jnp.where(coli == 21, v1 * inv,
                                        jnp.where(coli == 22, v2 * inv, 0.0))))
    out_ref[...] = out


def _tc_geom(pos_s, pos_d):
    BE = 640
    g = EP // BE
    return pl.pallas_call(
        _geom_body,
        grid=(g,),
        in_specs=[pl.BlockSpec((BE, 128), lambda i: (i, 0)),
                  pl.BlockSpec((BE, 128), lambda i: (i, 0))],
        out_specs=pl.BlockSpec((BE, 128), lambda i: (i, 0)),
        out_shape=jax.ShapeDtypeStruct((EP, 128), F32),
    )(pos_s, pos_d)


def _phi_body(x_ref, w1_ref, b1_ref, w2_ref, b2_ref, out_ref):
    h = jnp.dot(x_ref[...], w1_ref[...], preferred_element_type=F32) + b1_ref[...]
    h = _silu(h)
    out_ref[...] = jnp.dot(h, w2_ref[...], preferred_element_type=F32) + b2_ref[...]


def _tc_phi(x, w1, b1, w2, b2):
    BN = 512
    g = NP // BN
    return pl.pallas_call(
        _phi_body,
        grid=(g,),
        in_specs=[pl.BlockSpec((BN, 128), lambda i: (i, 0)),
                  pl.BlockSpec((128, 128), lambda i: (0, 0)),
                  pl.BlockSpec((1, 128), lambda i: (0, 0)),
                  pl.BlockSpec((128, 384), lambda i: (0, 0)),
                  pl.BlockSpec((1, 384), lambda i: (0, 0))],
        out_specs=pl.BlockSpec((BN, 384), lambda i: (i, 0)),
        out_shape=jax.ShapeDtypeStruct((NP, 384), F32),
    )(x, w1.reshape(1, -1) if w1.ndim == 1 else w1, b1.reshape(1, -1), w2, b2.reshape(1, -1))


def _msg_body_first(rbfd_ref, phis_ref, wr_ref, br_ref,
                    ms_ref, d0_ref, d1_ref, d2_ref):
    rbfd = rbfd_ref[...]
    W = jnp.dot(rbfd, wr_ref[...], preferred_element_type=F32) + br_ref[...]
    m = phis_ref[...] * W
    mvv = m[:, 128:256]
    mvr = m[:, 256:384]
    ms_ref[...] = m[:, 0:128]
    del mvv
    d0_ref[...] = rbfd[:, 20:21] * mvr
    d1_ref[...] = rbfd[:, 21:22] * mvr
    d2_ref[...] = rbfd[:, 22:23] * mvr


def _msg_body(rbfd_ref, phis_ref, vs_ref, wr_ref, br_ref,
              ms_ref, d0_ref, d1_ref, d2_ref):
    rbfd = rbfd_ref[...]
    W = jnp.dot(rbfd, wr_ref[...], preferred_element_type=F32) + br_ref[...]
    m = phis_ref[...] * W
    mvv = m[:, 128:256]
    mvr = m[:, 256:384]
    vs = vs_ref[...]
    ms_ref[...] = m[:, 0:128]
    d0_ref[...] = vs[:, 0:128] * mvv + rbfd[:, 20:21] * mvr
    d1_ref[...] = vs[:, 128:256] * mvv + rbfd[:, 21:22] * mvr
    d2_ref[...] = vs[:, 256:384] * mvv + rbfd[:, 22:23] * mvr


def _tc_msg(rbfd, phis, vs, wr, br):
    BE = 640
    g = EP // BE
    outs = [jax.ShapeDtypeStruct((EP, 128), F32)] * 4
    ospec = [pl.BlockSpec((BE, 128), lambda i: (i, 0))] * 4
    edge_spec = pl.BlockSpec((BE, 128), lambda i: (i, 0))
    big_spec = pl.BlockSpec((BE, 384), lambda i: (i, 0))
    wspec = [pl.BlockSpec((128, 384), lambda i: (0, 0)),
             pl.BlockSpec((1, 384), lambda i: (0, 0))]
    if vs is None:
        return pl.pallas_call(
            _msg_body_first,
            grid=(g,),
            in_specs=[edge_spec, big_spec] + wspec,
            out_specs=ospec,
            out_shape=outs,
        )(rbfd, phis, wr, br.reshape(1, -1))
    return pl.pallas_call(
        _msg_body,
        grid=(g,),
        in_specs=[edge_spec, big_spec, big_spec] + wspec,
        out_specs=ospec,
        out_shape=outs,
    )(rbfd, phis, vs, wr, br.reshape(1, -1))


def _upd_body(has_v, x_ref, v_ref, pms_ref, p0_ref, p1_ref, p2_ref,
              U_ref, V_ref, uw1_ref, ub1_ref, uw2_ref, ub2_ref,
              xo_ref, vo_ref):
    pms = pms_ref[...]
    x1 = x_ref[...] + pms[0] + pms[1]
    U = U_ref[...]
    Vm = V_ref[...]
    vks, uvs, vvs = [], [], []
    for k, pref in enumerate((p0_ref, p1_ref, p2_ref)):
        pk = pref[...]
        vk = pk[0] + pk[1]
        if has_v:
            vk = vk + v_ref[:, 128 * k:128 * (k + 1)]
        vks.append(vk)
        uvs.append(jnp.dot(vk, U, preferred_element_type=F32))
        vvs.append(jnp.dot(vk, Vm, preferred_element_type=F32))
    vvn = jnp.sqrt(vvs[0] * vvs[0] + vvs[1] * vvs[1] + vvs[2] * vvs[2] + 1e-12)
    cat = jnp.concatenate([x1, vvn], axis=1)
    a = _silu(jnp.dot(cat, uw1_ref[...], preferred_element_type=F32) + ub1_ref[...])
    a = jnp.dot(a, uw2_ref[...], preferred_element_type=F32) + ub2_ref[...]
    ass = a[:, 0:128]
    asv = a[:, 128:256]
    avv = a[:, 256:384]
    dot = uvs[0] * vvs[0] + uvs[1] * vvs[1] + uvs[2] * vvs[2]
    xo_ref[...] = x1 + ass + asv * dot
    vo_ref[...] = jnp.concatenate([vks[k] + avv * uvs[k] for k in range(3)], axis=1)


def _tc_update(x, v, pms, p0, p1, p2, U, V, uw1, ub1, uw2, ub2):
    BN = 512
    g = NP // BN
    pspec = pl.BlockSpec((2, BN, 128), lambda i: (0, i, 0))
    has_v = v is not None
    in_specs = [pl.BlockSpec((BN, 128), lambda i: (i, 0)),
                pl.BlockSpec((BN, 384), lambda i: (i, 0)),
                pspec, pspec, pspec, pspec,
                pl.BlockSpec((128, 128), lambda i: (0, 0)),
                pl.BlockSpec((128, 128), lambda i: (0, 0)),
                pl.BlockSpec((256, 128), lambda i: (0, 0)),
                pl.BlockSpec((1, 128), lambda i: (0, 0)),
                pl.BlockSpec((128, 384), lambda i: (0, 0)),
                pl.BlockSpec((1, 384), lambda i: (0, 0))]
    if not has_v:
        v = jnp.zeros((NP, 384), F32)
    return pl.pallas_call(
        functools.partial(_upd_body, has_v),
        grid=(g,),
        in_specs=in_specs,
        out_specs=[pl.BlockSpec((BN, 128), lambda i: (i, 0)),
                   pl.BlockSpec((BN, 384), lambda i: (i, 0))],
        out_shape=[jax.ShapeDtypeStruct((NP, 128), F32),
                   jax.ShapeDtypeStruct((NP, 384), F32)],
    )(x, v, pms, p0, p1, p2, U, V, uw1, ub1.reshape(1, -1), uw2, ub2.reshape(1, -1))


def _out_body(x_ref, b_ref, w1_ref, b1_ref, w2_ref, b2_ref, o_ref):
    i = pl.program_id(0)
    h = _silu(jnp.dot(x_ref[...], w1_ref[...], preferred_element_type=F32) + b1_ref[...])
    no = jnp.dot(h, w2_ref[...], preferred_element_type=F32) + b2_ref[...]
    bb = b_ref[...]
    gids = lax.broadcasted_iota(jnp.int32, (NG, bb.shape[1]), 0)
    oh = (gids == bb).astype(F32)
    contrib = jnp.dot(oh, no, preferred_element_type=F32)

    @pl.when(i == 0)
    def _():
        o_ref[...] = jnp.zeros_like(o_ref)

    o_ref[...] += contrib


def _tc_out(x, batch_row, w1, b1, w2, b2):
    BN = 512
    g = NP // BN
    return pl.pallas_call(
        _out_body,
        grid=(g,),
        in_specs=[pl.BlockSpec((BN, 128), lambda i: (i, 0)),
                  pl.BlockSpec((1, BN), lambda i: (0, i)),
                  pl.BlockSpec((128, HID), lambda i: (0, 0)),
                  pl.BlockSpec((1, HID), lambda i: (0, 0)),
                  pl.BlockSpec((HID, 1), lambda i: (0, 0)),
                  pl.BlockSpec((1, 1), lambda i: (0, 0))],
        out_specs=pl.BlockSpec((NG, 1), lambda i: (0, 0)),
        out_shape=jax.ShapeDtypeStruct((NG, 1), F32),
    )(x, batch_row, w1, b1.reshape(1, -1), w2, b2.reshape(1, -1))


def kernel(at_no, pos, edge_index, batch, params):
    src = jnp.pad(edge_index[0].astype(jnp.int32), (0, EP - E))
    dst = jnp.pad(edge_index[1].astype(jnp.int32), (0, EP - E),
                  constant_values=NP - 1)
    at_pad = jnp.pad(at_no.astype(jnp.int32), (0, NP - N))
    pos_p = jnp.pad(pos.astype(F32), ((0, NP - N), (0, 125)))
    batch_row = jnp.pad(batch.astype(jnp.int32), (0, NP - N),
                        constant_values=NG).reshape(1, NP)
    zeros_acc = jnp.zeros((NP, 128), F32)

    gather_embed = _sc_gather(120, 128, NP, 64)
    gather_pos = _sc_gather(NP, 128, EP, 128)
    gather_big = _sc_gather(NP, 384, EP, 64)
    scatter = _sc_scatter_add(EP, 128)

    x = gather_embed(params["embed"], at_pad)
    pos_s = gather_pos(pos_p, src)
    pos_d = gather_pos(pos_p, dst)
    rbfd = _tc_geom(pos_s, pos_d)

    v = None
    for blk in params["blocks"]:
        wrbf_pad = jnp.zeros((128, 3 * D), F32).at[:NB].set(blk["wrbf"])
        phi = _tc_phi(x, blk["mw1"], blk["mb1"], blk["mw2"], blk["mb2"])
        phi_s = gather_big(phi, src)
        v_s = gather_big(v, src) if v is not None else None
        ms, d0, d1, d2 = _tc_msg(rbfd, phi_s, v_s, wrbf_pad, blk["brbf"])
        pms = scatter(ms, dst, zeros_acc)
        pd0 = scatter(d0, dst, zeros_acc)
        pd1 = scatter(d1, dst, zeros_acc)
        pd2 = scatter(d2, dst, zeros_acc)
        x, v = _tc_update(x, v, pms, pd0, pd1, pd2,
                          blk["U"], blk["V"], blk["uw1"], blk["ub1"],
                          blk["uw2"], blk["ub2"])

    return _tc_out(x, batch_row, params["ow1"], params["ob1"],
                   params["ow2"], params["ob2"])
